# baseline (speedup 1.0000x reference)
import functools

import jax
import jax.numpy as jnp
from jax.experimental import pallas as pl
from jax.experimental.pallas import tpu as pltpu

_N = 262144
_K = 512
_D = 32
_BN = 2048


def _body(embs_ref, centers_ref, ids_ref, loss_ref):
    i = pl.program_id(0)
    e = embs_ref[...]
    c = centers_ref[...]
    cross = jax.lax.dot_general(
        e, c, (((1,), (1,)), ((), ())),
        preferred_element_type=jnp.float32)
    e2 = jnp.sum(e * e, axis=1, keepdims=True)
    c2 = jnp.sum(c * c, axis=1)[None, :]
    d2 = jnp.maximum(e2 - 2.0 * cross + c2, 0.0)
    m = jnp.min(d2, axis=1, keepdims=True)
    iota = jax.lax.broadcasted_iota(jnp.int32, d2.shape, 1)
    ids = jnp.min(jnp.where(d2 == m, iota, _K), axis=1)
    ids_ref[...] = ids

    @pl.when(i == 0)
    def _():
        loss_ref[0, 0] = 0.0

    loss_ref[0, 0] += jnp.sum(m)


@jax.jit
def _cluster(embs, centers):
    grid = _N // _BN
    ids, loss = pl.pallas_call(
        _body,
        grid=(grid,),
        in_specs=[
            pl.BlockSpec((_BN, _D), lambda i: (i, 0)),
            pl.BlockSpec((_K, _D), lambda i: (0, 0)),
        ],
        out_specs=[
            pl.BlockSpec((_BN,), lambda i: (i,)),
            pl.BlockSpec((1, 1), lambda i: (0, 0), memory_space=pltpu.SMEM),
        ],
        out_shape=[
            jax.ShapeDtypeStruct((_N,), jnp.int32),
            jax.ShapeDtypeStruct((1, 1), jnp.float32),
        ],
    )(embs, centers)
    return ids, loss[0, 0]


def kernel(embs, centers):
    ids, loss = _cluster(embs, centers)
    return (centers, ids, loss)


# speedup vs baseline: 1.8141x; 1.8141x over previous
import functools

import jax
import jax.numpy as jnp
from jax.experimental import pallas as pl
from jax.experimental.pallas import tpu as pltpu

_N = 262144
_K = 512
_D = 32
_BN = 2048


def _body(e_ref, cm2_ref, c2_ref, ids_ref, loss_ref):
    i = pl.program_id(0)
    e = e_ref[...]
    cm2 = cm2_ref[...]
    s = jax.lax.dot_general(
        cm2, e, (((1,), (1,)), ((), ())),
        preferred_element_type=jnp.float32)
    s = s + c2_ref[...]
    m = jnp.min(s, axis=0, keepdims=True)
    iota = jax.lax.broadcasted_iota(jnp.int32, s.shape, 0)
    ids = jnp.min(jnp.where(s == m, iota, _K), axis=0)
    ids_ref[...] = ids

    part = jnp.sum(e * e) + jnp.sum(m)

    @pl.when(i == 0)
    def _():
        loss_ref[0, 0] = 0.0

    loss_ref[0, 0] += part


@jax.jit
def _cluster(embs, centers):
    cm2 = -2.0 * centers
    c2 = jnp.sum(centers * centers, axis=1, keepdims=True)
    grid = _N // _BN
    ids, loss = pl.pallas_call(
        _body,
        grid=(grid,),
        in_specs=[
            pl.BlockSpec((_BN, _D), lambda i: (i, 0)),
            pl.BlockSpec((_K, _D), lambda i: (0, 0)),
            pl.BlockSpec((_K, 1), lambda i: (0, 0)),
        ],
        out_specs=[
            pl.BlockSpec((_BN,), lambda i: (i,)),
            pl.BlockSpec((1, 1), lambda i: (0, 0), memory_space=pltpu.SMEM),
        ],
        out_shape=[
            jax.ShapeDtypeStruct((_N,), jnp.int32),
            jax.ShapeDtypeStruct((1, 1), jnp.float32),
        ],
    )(embs, cm2, c2)
    return ids, loss[0, 0]


def kernel(embs, centers):
    ids, loss = _cluster(embs, centers)
    return (centers, ids, loss)


# speedup vs baseline: 2.5807x; 1.4226x over previous
import functools

import jax
import jax.numpy as jnp
from jax.experimental import pallas as pl
from jax.experimental.pallas import tpu as pltpu

_N = 262144
_K = 512
_D = 32
_BN = 2048


def _body(ea_ref, ca_ref, ids_ref, loss_ref):
    i = pl.program_id(0)
    ea = ea_ref[...]
    ca = ca_ref[...]
    s = jax.lax.dot_general(
        ca, ea, (((1,), (0,)), ((), ())),
        preferred_element_type=jnp.float32)
    m = jnp.min(s, axis=0, keepdims=True)
    iota = jax.lax.broadcasted_iota(jnp.int32, s.shape, 0)
    ids = jnp.min(jnp.where(s == m, iota, _K), axis=0)
    ids_ref[...] = ids

    e = ea[:_D, :]
    part = jnp.sum(e * e) + jnp.sum(m)

    @pl.when(i == 0)
    def _():
        loss_ref[0, 0] = 0.0

    loss_ref[0, 0] += part


@jax.jit
def _cluster(embs, centers):
    c2 = jnp.sum(centers * centers, axis=1, keepdims=True)
    ca = jnp.concatenate([-2.0 * centers, c2], axis=1)
    eaT = jnp.concatenate(
        [embs.T, jnp.ones((1, _N), jnp.float32)], axis=0)
    grid = _N // _BN
    ids, loss = pl.pallas_call(
        _body,
        grid=(grid,),
        in_specs=[
            pl.BlockSpec((_D + 1, _BN), lambda i: (0, i)),
            pl.BlockSpec((_K, _D + 1), lambda i: (0, 0)),
        ],
        out_specs=[
            pl.BlockSpec((_BN,), lambda i: (i,)),
            pl.BlockSpec((1, 1), lambda i: (0, 0), memory_space=pltpu.SMEM),
        ],
        out_shape=[
            jax.ShapeDtypeStruct((_N,), jnp.int32),
            jax.ShapeDtypeStruct((1, 1), jnp.float32),
        ],
    )(eaT, ca)
    return ids, loss[0, 0]


def kernel(embs, centers):
    ids, loss = _cluster(embs, centers)
    return (centers, ids, loss)
